# baseline (speedup 1.0000x reference)
import jax
import jax.numpy as jnp
from jax import lax
from jax.experimental import pallas as pl
from jax.experimental.pallas import tpu as pltpu
from jax.experimental.pallas import tpu_sc as plsc

B = 16384
VOCAB = 1000000
D = 32
NCTX = 5
FLAT = VOCAB * D

NC = 2
NS = 16
NW = NC * NS
BPW = B // NW
CPW = BPW * NCTX
CHB = 64
NCH = BPW // CHB
GRP = CHB // 16
ROWS = CHB * (1 + NCTX)
ELEM = ROWS * D
NSEG = ELEM // 128
CROW = CHB * NCTX
TEL = CHB * D
TSEG = TEL // 128


def _body(tt_hbm, tidx_hbm, ct_hbm, cidx_hbm, out_hbm,
          tidx_v, cidx_v, row_v, idx_v, g_v, out_v, sem):
  cid = lax.axis_index("c")
  sid = lax.axis_index("s")
  wid = cid * NS + sid

  pltpu.sync_copy(tidx_hbm.at[pl.ds(wid * BPW, BPW)], tidx_v)
  pltpu.sync_copy(cidx_hbm.at[pl.ds(wid * CPW, CPW)], cidx_v)

  iota16 = lax.broadcasted_iota(jnp.int32, (16,), 0)

  def build(ch, slot):
    cb = ch * CHB
    for j in range(GRP):
      row_v[pl.ds(j * 16, 16)] = tidx_v[pl.ds(cb + j * 16, 16)]
    for n in range(NCTX):
      for j in range(GRP):
        row_v[pl.ds(CHB + n * CHB + j * 16, 16)] = plsc.load_gather(
            cidx_v, [(cb + j * 16 + iota16) * NCTX + n])

    def per_dim(d, carry):
      off = d * VOCAB
      tb = slot * ELEM + d * CHB
      cb2 = slot * ELEM + TEL + d * CROW
      for g in range(GRP):
        idx_v[pl.ds(tb + g * 16, 16)] = row_v[pl.ds(g * 16, 16)] + off
      for g in range(CROW // 16):
        idx_v[pl.ds(cb2 + g * 16, 16)] = (
            row_v[pl.ds(CHB + g * 16, 16)] + off)
      return carry

    lax.fori_loop(0, D, per_dim, 0)

  def fire(slot):
    for s in range(NSEG):
      src = tt_hbm if s < TSEG else ct_hbm
      e = slot * ELEM + s * 128
      pltpu.async_copy(src.at[idx_v.at[pl.ds(e, 128)]],
                       g_v.at[pl.ds(e, 128)], sem.at[slot])

  def drain(slot):
    pltpu.make_async_copy(tt_hbm.at[pl.ds(0, TSEG * 128)],
                          g_v.at[pl.ds(slot * ELEM, TSEG * 128)],
                          sem.at[slot]).wait()
    pltpu.make_async_copy(ct_hbm.at[pl.ds(0, ELEM - TSEG * 128)],
                          g_v.at[pl.ds(slot * ELEM + TSEG * 128,
                                       ELEM - TSEG * 128)],
                          sem.at[slot]).wait()

  zero16 = jnp.zeros((16,), jnp.float32)

  def compute(ch, slot):
    for j in range(GRP):
      accs = [zero16] * NCTX
      for d in range(D):
        t = g_v[pl.ds(slot * ELEM + d * CHB + j * 16, 16)]
        ce = slot * ELEM + TEL + d * CROW
        for n in range(NCTX):
          c = g_v[pl.ds(ce + n * CHB + j * 16, 16)]
          accs[n] = accs[n] + c * t
      for n in range(NCTX):
        out_v[pl.ds(n * BPW + ch * CHB + j * 16, 16)] = accs[n]

  build(0, 0)
  fire(0)

  def body(ch, carry):
    slot = ch & 1

    @pl.when(ch < NCH - 1)
    def _():
      build(ch + 1, 1 - slot)
      fire(1 - slot)

    drain(slot)
    compute(ch, slot)
    return carry

  lax.fori_loop(0, NCH, body, 0)

  pltpu.sync_copy(out_v, out_hbm.at[pl.ds(wid * CPW, CPW)])


@jax.jit
def kernel(target, context, target_table, context_table):
  tidx = target.reshape(B)
  cidx = context.reshape(B * NCTX)
  tt_flat = target_table.T.reshape(FLAT)
  ct_flat = context_table.T.reshape(FLAT)

  mesh = plsc.VectorSubcoreMesh(core_axis_name="c", subcore_axis_name="s")
  run = pl.kernel(
      _body,
      out_type=jax.ShapeDtypeStruct((NW * CPW,), jnp.float32),
      mesh=mesh,
      scratch_types=[
          pltpu.VMEM((BPW,), jnp.int32),
          pltpu.VMEM((CPW,), jnp.int32),
          pltpu.VMEM((ROWS,), jnp.int32),
          pltpu.VMEM((2 * ELEM,), jnp.int32),
          pltpu.VMEM((2 * ELEM,), jnp.float32),
          pltpu.VMEM((CPW,), jnp.float32),
          pltpu.SemaphoreType.DMA((2,)),
      ],
      compiler_params=pltpu.CompilerParams(needs_layout_passes=False),
  )
  out_flat = run(tt_flat, tidx, ct_flat, cidx)
  return out_flat.reshape(NW, NCTX, BPW).transpose(0, 2, 1).reshape(B, NCTX)
